# baseline (speedup 1.0000x reference)
import functools

import jax
import jax.numpy as jnp
from jax import lax
from jax.experimental import pallas as pl
from jax.experimental.pallas import tpu as pltpu
from jax.experimental.pallas import tpu_sc as plsc

NPTS = 498432
NBINS = 129600
TRASH = 960
ROWS = NBINS + TRASH
CCHUNK = 8
NCHUNK = 10

RA, CA = 128, 3894

PT_TILE = NPTS // 16
NWIN = 11
WSZ = PT_TILE // NWIN
ZROWS = 1020
STRIPE = ROWS // 16


def _voxel_body(gx_ref, gy_ref, gz_ref, idx_ref):
    xf = (gx_ref[...] - (-54.0)) / 0.3
    yf = (gy_ref[...] - (-54.0)) / 0.3
    zf = (gz_ref[...] - (-10.0)) / 20.0
    kept = (
        (xf >= 0.0) & (xf < 360.0)
        & (yf >= 0.0) & (yf < 360.0)
        & (zf >= 0.0) & (zf < 1.0)
    )
    ix = xf.astype(jnp.int32)
    iy = yf.astype(jnp.int32)
    flat = ix * 360 + iy
    col = lax.broadcasted_iota(jnp.int32, (8, CA), 1)
    idx_ref[...] = jnp.where(kept, flat, NBINS + col % TRASH)


def _repack_body(x_ref, o_ref):
    xi = x_ref[...].reshape(1408, 80)
    for c in range(NCHUNK):
        o_ref[c] = xi[:, CCHUNK * c:CCHUNK * (c + 1)]


def _scatter_body(x_hbm, idx_hbm, out_hbm, acc, xbuf, ibuf, zbuf):
    c = lax.axis_index("c")
    s = lax.axis_index("s")

    def _zrow(r, carry):
        zbuf[r, :] = jnp.zeros((CCHUNK,), jnp.float32)
        return carry

    lax.fori_loop(0, ZROWS, _zrow, 0)

    for k in range(5):
        chunk = 2 * k + c

        for j in range(8):
            pltpu.sync_copy(zbuf, acc.at[pl.ds(s * STRIPE + j * ZROWS, ZROWS)])
        plsc.subcore_barrier()

        for w in range(NWIN):
            base = s * PT_TILE + w * WSZ
            pltpu.sync_copy(idx_hbm.at[pl.ds(base, WSZ)], ibuf)
            pltpu.sync_copy(x_hbm.at[chunk, pl.ds(base, WSZ), :], xbuf)
            pltpu.sync_copy(xbuf, acc.at[ibuf], add=True)
        plsc.subcore_barrier()

        wo = jnp.minimum(s * 8104, NBINS - 8104)
        pltpu.sync_copy(
            acc.at[pl.ds(wo, 8104)],
            out_hbm.at[chunk, pl.ds(wo, 8104)],
        )
        plsc.subcore_barrier()


def _assemble_body(a_ref, o_ref):
    eye = jnp.eye(CCHUNK, dtype=jnp.float32)
    dn = (((1,), (1,)), ((), ()))
    ta = lax.dot_general(eye, a_ref[0], dn, preferred_element_type=jnp.float32)
    o_ref[...] = ta.reshape(CCHUNK, 8, 2025)


def kernel(geom_feats, x):
    gflat = geom_feats.reshape(NPTS, 3)
    gxa = gflat[:, 0].reshape(RA, CA)
    gya = gflat[:, 1].reshape(RA, CA)
    gza = gflat[:, 2].reshape(RA, CA)

    idx2d = pl.pallas_call(
        _voxel_body,
        grid=(RA // 8,),
        in_specs=[pl.BlockSpec((8, CA), lambda i: (i, 0))] * 3,
        out_specs=pl.BlockSpec((8, CA), lambda i: (i, 0)),
        out_shape=jax.ShapeDtypeStruct((RA, CA), jnp.int32),
    )(gxa, gya, gza)
    idx = idx2d.reshape(NPTS)

    xs = pl.pallas_call(
        _repack_body,
        grid=(6, 59),
        in_specs=[
            pl.BlockSpec(
                (1, 1, 2, 16, 44, 80), lambda n, d: (0, n, d, 0, 0, 0)
            )
        ],
        out_specs=pl.BlockSpec(
            (NCHUNK, 1408, CCHUNK), lambda n, d: (0, n * 59 + d, 0)
        ),
        out_shape=jax.ShapeDtypeStruct((NCHUNK, NPTS, CCHUNK), jnp.float32),
    )(x)

    mesh = plsc.VectorSubcoreMesh(core_axis_name="c", subcore_axis_name="s")
    scatter = functools.partial(
        pl.kernel,
        out_type=jax.ShapeDtypeStruct((NCHUNK, NBINS, CCHUNK), jnp.float32),
        mesh=mesh,
        compiler_params=pltpu.CompilerParams(use_tc_tiling_on_sc=False),
        scratch_types=[
            pltpu.VMEM_SHARED((ROWS, CCHUNK), jnp.float32),
            pltpu.VMEM((WSZ, CCHUNK), jnp.float32),
            pltpu.VMEM((WSZ,), jnp.int32),
            pltpu.VMEM((ZROWS, CCHUNK), jnp.float32),
        ],
    )(_scatter_body)
    pooled = scatter(xs, idx)

    out = pl.pallas_call(
        _assemble_body,
        grid=(NCHUNK, 8),
        in_specs=[
            pl.BlockSpec((1, 16200, CCHUNK), lambda j, b: (j, b, 0)),
        ],
        out_specs=pl.BlockSpec((CCHUNK, 8, 2025), lambda j, b: (j, b, 0)),
        out_shape=jax.ShapeDtypeStruct((80, 64, 2025), jnp.float32),
    )(pooled)

    return out.reshape(1, 80, 360, 360)


# speedup vs baseline: 1.1765x; 1.1765x over previous
import functools

import jax
import jax.numpy as jnp
from jax import lax
from jax.experimental import pallas as pl
from jax.experimental.pallas import tpu as pltpu
from jax.experimental.pallas import tpu_sc as plsc

NPTS = 498432
NBINS = 129600
TRASH = 960
ROWS = NBINS + TRASH
CCHUNK = 8
NCHUNK = 10

PT_TILE = NPTS // 16
NWIN = 11
WSZ = PT_TILE // NWIN
XROWS = WSZ * CCHUNK // 128
ZROWS = 1020
STRIPE = ROWS // 16


def _voxel_body(g_ref, idx_ref):
    g = g_ref[...].reshape(1408, 3)
    xf = (g[:, 0] - (-54.0)) / 0.3
    yf = (g[:, 1] - (-54.0)) / 0.3
    zf = (g[:, 2] - (-10.0)) / 20.0
    kept = (
        (xf >= 0.0) & (xf < 360.0)
        & (yf >= 0.0) & (yf < 360.0)
        & (zf >= 0.0) & (zf < 1.0)
    )
    ix = xf.astype(jnp.int32)
    iy = yf.astype(jnp.int32)
    flat = ix * 360 + iy
    pos = lax.broadcasted_iota(jnp.int32, (1408,), 0)
    idx = jnp.where(kept, flat, NBINS + pos % TRASH)
    idx_ref[...] = idx.reshape(1, 11, 128)


def _idxflat_body(i_ref, o_ref):
    o_ref[pl.ds(0, 3894), :] = i_ref[...].reshape(3894, 128)


def _unpad_body(x_ref, o_ref):
    o_ref[...] = x_ref[...].reshape(1408, 80)


def _scatter_body(x_hbm, idx_hbm, out_hbm, acc, xb0, xb1, ib0, ib1, zbuf,
                  sx0, sx1, si0, si1):
    c = lax.axis_index("c")
    s = lax.axis_index("s")
    xbufs = (xb0, xb1)
    ibufs = (ib0, ib1)
    sxs = (sx0, sx1)
    sis = (si0, si1)

    def _zrow(r, carry):
        zbuf[r, :] = jnp.zeros((CCHUNK,), jnp.float32)
        return carry

    lax.fori_loop(0, ZROWS, _zrow, 0)

    def _issue(chunk, w, p):
        base = s * PT_TILE + w * WSZ
        di = pltpu.async_copy(idx_hbm.at[pl.ds(base, WSZ)], ibufs[p], sis[p])
        dx = pltpu.async_copy(
            x_hbm.at[pl.ds(base, WSZ), pl.ds(chunk * CCHUNK, CCHUNK)],
            xbufs[p],
            sxs[p],
        )
        return di, dx

    for k in range(5):
        chunk = 2 * k + c

        for j in range(8):
            pltpu.sync_copy(zbuf, acc.at[pl.ds(s * STRIPE + j * ZROWS, ZROWS)])
        plsc.subcore_barrier()

        pend = _issue(chunk, 0, 0)
        for w in range(NWIN):
            p = w & 1
            di, dx = pend
            di.wait()
            dx.wait()
            if w + 1 < NWIN:
                pend = _issue(chunk, w + 1, 1 - p)
            pltpu.sync_copy(xbufs[p], acc.at[ibufs[p]], add=True)
        plsc.subcore_barrier()

        wo = jnp.minimum(s * 8104, NBINS - 8104)
        pltpu.sync_copy(
            acc.at[pl.ds(wo, 8104)],
            out_hbm.at[chunk, pl.ds(wo, 8104)],
        )
        plsc.subcore_barrier()


def _assemble_body(a_ref, o_ref):
    eye = jnp.eye(CCHUNK, dtype=jnp.float32)
    dn = (((1,), (1,)), ((), ()))
    a = a_ref[0]
    for xr in range(8):
        o_ref[0, :, xr, :] = lax.dot_general(
            eye, a[xr], dn, preferred_element_type=jnp.float32
        )


def kernel(geom_feats, x):
    idx2d = pl.pallas_call(
        _voxel_body,
        grid=(6, 59),
        in_specs=[
            pl.BlockSpec(
                (1, 1, 2, 16, 44, 3), lambda n, d: (0, n, d, 0, 0, 0)
            )
        ],
        out_specs=pl.BlockSpec((1, 11, 128), lambda n, d: (n * 59 + d, 0, 0)),
        out_shape=jax.ShapeDtypeStruct((354, 11, 128), jnp.int32),
    )(geom_feats)
    idxflat = pl.pallas_call(
        _idxflat_body,
        grid=(1,),
        in_specs=[pl.BlockSpec((354, 11, 128), lambda i: (0, 0, 0))],
        out_specs=pl.BlockSpec((3896, 128), lambda i: (0, 0)),
        out_shape=jax.ShapeDtypeStruct((3896, 128), jnp.int32),
    )(idx2d)
    idx = idxflat.reshape(3896 * 128)

    xs = pl.pallas_call(
        _unpad_body,
        grid=(6, 59),
        in_specs=[
            pl.BlockSpec(
                (1, 1, 2, 16, 44, 80), lambda n, d: (0, n, d, 0, 0, 0)
            )
        ],
        out_specs=pl.BlockSpec((1408, 80), lambda n, d: (n * 59 + d, 0)),
        out_shape=jax.ShapeDtypeStruct((NPTS, 80), jnp.float32),
    )(x)

    mesh = plsc.VectorSubcoreMesh(core_axis_name="c", subcore_axis_name="s")
    scatter = functools.partial(
        pl.kernel,
        out_type=jax.ShapeDtypeStruct((NCHUNK, NBINS, CCHUNK), jnp.float32),
        mesh=mesh,
        compiler_params=pltpu.CompilerParams(use_tc_tiling_on_sc=False),
        scratch_types=[
            pltpu.VMEM_SHARED((ROWS, CCHUNK), jnp.float32),
            pltpu.VMEM((WSZ, CCHUNK), jnp.float32),
            pltpu.VMEM((WSZ, CCHUNK), jnp.float32),
            pltpu.VMEM((WSZ,), jnp.int32),
            pltpu.VMEM((WSZ,), jnp.int32),
            pltpu.VMEM((ZROWS, CCHUNK), jnp.float32),
            pltpu.SemaphoreType.DMA,
            pltpu.SemaphoreType.DMA,
            pltpu.SemaphoreType.DMA,
            pltpu.SemaphoreType.DMA,
        ],
    )(_scatter_body)
    pooled = scatter(xs, idx)

    out = pl.pallas_call(
        _assemble_body,
        grid=(NCHUNK, 45),
        in_specs=[
            pl.BlockSpec((1, 8, 360, CCHUNK), lambda j, r: (j, r, 0, 0)),
        ],
        out_specs=pl.BlockSpec(
            (1, CCHUNK, 8, 360), lambda j, r: (0, j, r, 0)
        ),
        out_shape=jax.ShapeDtypeStruct((1, 80, 360, 360), jnp.float32),
    )(pooled.reshape(NCHUNK, 360, 360, CCHUNK))

    return out


# speedup vs baseline: 1.3802x; 1.1731x over previous
import functools

import jax
import jax.numpy as jnp
from jax import lax
from jax.experimental import pallas as pl
from jax.experimental.pallas import tpu as pltpu
from jax.experimental.pallas import tpu_sc as plsc

NPTS = 498432
NBINS = 129600
TRASH = 960
ROWS = NBINS + TRASH
CCHUNK = 8
NCHUNK = 10

PT_TILE = NPTS // 16
NWIN = 11
WSZ = PT_TILE // NWIN
XROWS = WSZ * CCHUNK // 128
ZROWS = 1020
STRIPE = ROWS // 16


def _voxel_body(g_ref, idx_ref):
    g = g_ref[...].reshape(1408, 3)
    xf = (g[:, 0] - (-54.0)) / 0.3
    yf = (g[:, 1] - (-54.0)) / 0.3
    zf = (g[:, 2] - (-10.0)) / 20.0
    kept = (
        (xf >= 0.0) & (xf < 360.0)
        & (yf >= 0.0) & (yf < 360.0)
        & (zf >= 0.0) & (zf < 1.0)
    )
    ix = xf.astype(jnp.int32)
    iy = yf.astype(jnp.int32)
    flat = ix * 360 + iy
    pos = lax.broadcasted_iota(jnp.int32, (1408,), 0)
    idx = jnp.where(kept, flat, NBINS + pos % TRASH)
    idx_ref[...] = idx.reshape(1, 11, 128)


def _idxflat_body(i_ref, o_ref):
    o_ref[pl.ds(0, 3894), :] = i_ref[...].reshape(3894, 128)


def _scatter_body(x_hbm, idx_hbm, out_hbm, acc, xb0, xb1, ib0, ib1, zbuf,
                  sx0, sx1, si0, si1):
    c = lax.axis_index("c")
    s = lax.axis_index("s")
    xbufs = (xb0, xb1)
    ibufs = (ib0, ib1)
    sxs = (sx0, sx1)
    sis = (si0, si1)

    def _zrow(r, carry):
        zbuf[r, :] = jnp.zeros((CCHUNK,), jnp.float32)
        return carry

    lax.fori_loop(0, ZROWS, _zrow, 0)

    def _issue(chunk, w, p):
        base = s * PT_TILE + w * WSZ
        di = pltpu.async_copy(idx_hbm.at[pl.ds(base, WSZ)], ibufs[p], sis[p])
        dx = pltpu.async_copy(
            x_hbm.at[pl.ds(base, WSZ), pl.ds(chunk * CCHUNK, CCHUNK)],
            xbufs[p],
            sxs[p],
        )
        return di, dx

    for k in range(5):
        chunk = 2 * k + c

        for j in range(8):
            pltpu.sync_copy(zbuf, acc.at[pl.ds(s * STRIPE + j * ZROWS, ZROWS)])
        plsc.subcore_barrier()

        pend = _issue(chunk, 0, 0)
        for w in range(NWIN):
            p = w & 1
            di, dx = pend
            di.wait()
            dx.wait()
            if w + 1 < NWIN:
                pend = _issue(chunk, w + 1, 1 - p)
            pltpu.sync_copy(xbufs[p], acc.at[ibufs[p]], add=True)
        plsc.subcore_barrier()

        wo = jnp.minimum(s * 8104, NBINS - 8104)
        pltpu.sync_copy(
            acc.at[pl.ds(wo, 8104)],
            out_hbm.at[chunk, pl.ds(wo, 8104)],
        )
        plsc.subcore_barrier()


def _assemble_body(a_ref, o_ref):
    eye = jnp.eye(CCHUNK, dtype=jnp.float32)
    dn = (((1,), (1,)), ((), ()))
    a = a_ref[0].reshape(8, 360, CCHUNK)
    for xr in range(8):
        o_ref[0, :, xr, :] = lax.dot_general(
            eye, a[xr], dn, preferred_element_type=jnp.float32
        )


def kernel(geom_feats, x):
    idx2d = pl.pallas_call(
        _voxel_body,
        grid=(6, 59),
        in_specs=[
            pl.BlockSpec(
                (1, 1, 2, 16, 44, 3), lambda n, d: (0, n, d, 0, 0, 0)
            )
        ],
        out_specs=pl.BlockSpec((1, 11, 128), lambda n, d: (n * 59 + d, 0, 0)),
        out_shape=jax.ShapeDtypeStruct((354, 11, 128), jnp.int32),
    )(geom_feats)
    idxflat = pl.pallas_call(
        _idxflat_body,
        grid=(1,),
        in_specs=[pl.BlockSpec((354, 11, 128), lambda i: (0, 0, 0))],
        out_specs=pl.BlockSpec((3896, 128), lambda i: (0, 0)),
        out_shape=jax.ShapeDtypeStruct((3896, 128), jnp.int32),
    )(idx2d)
    idx = idxflat.reshape(3896 * 128)

    xs = x.reshape(NPTS, 80)

    mesh = plsc.VectorSubcoreMesh(core_axis_name="c", subcore_axis_name="s")
    scatter = functools.partial(
        pl.kernel,
        out_type=jax.ShapeDtypeStruct((NCHUNK, NBINS, CCHUNK), jnp.float32),
        mesh=mesh,
        compiler_params=pltpu.CompilerParams(use_tc_tiling_on_sc=False),
        scratch_types=[
            pltpu.VMEM_SHARED((ROWS, CCHUNK), jnp.float32),
            pltpu.VMEM((WSZ, CCHUNK), jnp.float32),
            pltpu.VMEM((WSZ, CCHUNK), jnp.float32),
            pltpu.VMEM((WSZ,), jnp.int32),
            pltpu.VMEM((WSZ,), jnp.int32),
            pltpu.VMEM((ZROWS, CCHUNK), jnp.float32),
            pltpu.SemaphoreType.DMA,
            pltpu.SemaphoreType.DMA,
            pltpu.SemaphoreType.DMA,
            pltpu.SemaphoreType.DMA,
        ],
    )(_scatter_body)
    pooled = scatter(xs, idx)

    out = pl.pallas_call(
        _assemble_body,
        grid=(NCHUNK, 45),
        in_specs=[
            pl.BlockSpec((1, 2880, CCHUNK), lambda j, r: (j, r, 0)),
        ],
        out_specs=pl.BlockSpec(
            (1, CCHUNK, 8, 360), lambda j, r: (0, j, r, 0)
        ),
        out_shape=jax.ShapeDtypeStruct((1, 80, 360, 360), jnp.float32),
    )(pooled)

    return out


# speedup vs baseline: 2.1820x; 1.5809x over previous
import functools

import jax
import jax.numpy as jnp
from jax import lax
from jax.experimental import pallas as pl
from jax.experimental.pallas import tpu as pltpu
from jax.experimental.pallas import tpu_sc as plsc

NPTS = 498432
NBINS = 129600
TRASH = 960
ROWS = NBINS + TRASH
CCHUNK = 8
NCHUNK = 10

PT_TILE = NPTS // 16
NWIN = 11
WSZ = PT_TILE // NWIN
XROWS = WSZ * CCHUNK // 128
ZROWS = 1020
STRIPE = ROWS // 16


def _voxel_body(g_ref, idx_ref):
    xf = (g_ref[0] - (-54.0)) / 0.3
    yf = (g_ref[1] - (-54.0)) / 0.3
    zf = (g_ref[2] - (-10.0)) / 20.0
    kept = (
        (xf >= 0.0) & (xf < 360.0)
        & (yf >= 0.0) & (yf < 360.0)
        & (zf >= 0.0) & (zf < 1.0)
    )
    ix = xf.astype(jnp.int32)
    iy = yf.astype(jnp.int32)
    flat = ix * 360 + iy
    pos = (
        lax.broadcasted_iota(jnp.int32, (8, 128), 0) * 128
        + lax.broadcasted_iota(jnp.int32, (8, 128), 1)
    )
    idx_ref[...] = jnp.where(kept, flat, NBINS + pos % TRASH)


def _scatter_body(x_hbm, idx_hbm, out_hbm, acc, xb0, xb1, ib0, ib1, zbuf,
                  sx0, sx1, si0, si1):
    c = lax.axis_index("c")
    s = lax.axis_index("s")
    xbufs = (xb0, xb1)
    ibufs = (ib0, ib1)
    sxs = (sx0, sx1)
    sis = (si0, si1)

    def _zrow(r, carry):
        zbuf[r, :] = jnp.zeros((CCHUNK,), jnp.float32)
        return carry

    lax.fori_loop(0, ZROWS, _zrow, 0)

    def _issue(chunk, w, p):
        base = s * PT_TILE + w * WSZ
        di = pltpu.async_copy(idx_hbm.at[pl.ds(base, WSZ)], ibufs[p], sis[p])
        dx = pltpu.async_copy(
            x_hbm.at[pl.ds(base, WSZ), pl.ds(chunk * CCHUNK, CCHUNK)],
            xbufs[p],
            sxs[p],
        )
        return di, dx

    for k in range(5):
        chunk = 2 * k + c

        for j in range(8):
            pltpu.sync_copy(zbuf, acc.at[pl.ds(s * STRIPE + j * ZROWS, ZROWS)])
        plsc.subcore_barrier()

        pend = _issue(chunk, 0, 0)
        for w in range(NWIN):
            p = w & 1
            di, dx = pend
            di.wait()
            dx.wait()
            if w + 1 < NWIN:
                pend = _issue(chunk, w + 1, 1 - p)
            pltpu.sync_copy(xbufs[p], acc.at[ibufs[p]], add=True)
        plsc.subcore_barrier()

        pltpu.sync_copy(
            acc.at[pl.ds(s * 8104, 8104)],
            out_hbm.at[chunk, pl.ds(s * 8104, 8104)],
        )
        plsc.subcore_barrier()


def _assemble_body(a_ref, o_ref):
    eye = jnp.eye(CCHUNK, dtype=jnp.float32)
    dn = (((1,), (1,)), ((), ()))
    a = a_ref[0].reshape(8, 360, CCHUNK)
    for xr in range(8):
        o_ref[0, :, xr, :] = lax.dot_general(
            eye, a[xr], dn, preferred_element_type=jnp.float32
        )


def kernel(geom_feats, x):
    gcols = geom_feats.reshape(NPTS, 3).T.reshape(3, 3894, 128)
    idx2d = pl.pallas_call(
        _voxel_body,
        grid=(487,),
        in_specs=[pl.BlockSpec((3, 8, 128), lambda i: (0, i, 0))],
        out_specs=pl.BlockSpec((8, 128), lambda i: (i, 0)),
        out_shape=jax.ShapeDtypeStruct((3896, 128), jnp.int32),
    )(gcols)
    idx = idx2d.reshape(3896 * 128)

    xs = x.reshape(NPTS, 80)

    mesh = plsc.VectorSubcoreMesh(core_axis_name="c", subcore_axis_name="s")
    scatter = functools.partial(
        pl.kernel,
        out_type=jax.ShapeDtypeStruct((NCHUNK, 129664, CCHUNK), jnp.float32),
        mesh=mesh,
        compiler_params=pltpu.CompilerParams(use_tc_tiling_on_sc=False),
        scratch_types=[
            pltpu.VMEM_SHARED((ROWS, CCHUNK), jnp.float32),
            pltpu.VMEM((WSZ, CCHUNK), jnp.float32),
            pltpu.VMEM((WSZ, CCHUNK), jnp.float32),
            pltpu.VMEM((WSZ,), jnp.int32),
            pltpu.VMEM((WSZ,), jnp.int32),
            pltpu.VMEM((ZROWS, CCHUNK), jnp.float32),
            pltpu.SemaphoreType.DMA,
            pltpu.SemaphoreType.DMA,
            pltpu.SemaphoreType.DMA,
            pltpu.SemaphoreType.DMA,
        ],
    )(_scatter_body)
    pooled = scatter(xs, idx)

    out = pl.pallas_call(
        _assemble_body,
        grid=(NCHUNK, 45),
        in_specs=[
            pl.BlockSpec((1, 2880, CCHUNK), lambda j, r: (j, r, 0)),
        ],
        out_specs=pl.BlockSpec(
            (1, CCHUNK, 8, 360), lambda j, r: (0, j, r, 0)
        ),
        out_shape=jax.ShapeDtypeStruct((1, 80, 360, 360), jnp.float32),
    )(pooled)

    return out
